# Initial kernel scaffold; baseline (speedup 1.0000x reference)
import jax
import jax.numpy as jnp
from jax.experimental import pallas as pl


def kernel(ctrl_tokens, embed_table, proj_w, proj_b):
    raise NotImplementedError("write your pallas kernel here")


# speedup vs baseline: 4.2567x; 4.2567x over previous
import functools

import jax
import jax.numpy as jnp
from jax import lax
from jax.experimental import pallas as pl
from jax.experimental.pallas import tpu as pltpu
from jax.experimental.pallas import tpu_sc as plsc

_NUM_CORES = 2
_NUM_SUBCORES = 16
_NW = _NUM_CORES * _NUM_SUBCORES
_LANES = 16

_CHUNK = 128


def _proj_body(x_ref, w_ref, b_ref, o_ref):
    o_ref[...] = (
        jnp.dot(x_ref[...], w_ref[...], preferred_element_type=jnp.float32)
        + b_ref[...]
    )


def _precompute_tables(embed_table, proj_w, proj_b, row_tile):
    v, d = embed_table.shape
    s = proj_w.shape[1] // d
    wt = proj_w.reshape(d, s, d).transpose(1, 2, 0).reshape(s * d, d)
    bias = (proj_b / s).reshape(1, d).astype(jnp.float32)
    n_row_tiles = v // row_tile
    return pl.pallas_call(
        _proj_body,
        grid=(n_row_tiles, s),
        in_specs=[
            pl.BlockSpec((row_tile, d), lambda k, j: (k, 0)),
            pl.BlockSpec((d, d), lambda k, j: (j, 0)),
            pl.BlockSpec((1, d), lambda k, j: (0, 0)),
        ],
        out_specs=pl.BlockSpec((row_tile, d), lambda k, j: (j * n_row_tiles + k, 0)),
        out_shape=jax.ShapeDtypeStruct((s * v, d), jnp.float32),
    )(embed_table, wt, bias)


def _gather_sum(flat_idx, tables, n_tokens, s, d):
    per_w = n_tokens // _NW
    n_chunks = per_w // _CHUNK
    vregs_per_tok = d // _LANES

    mesh = plsc.VectorSubcoreMesh(
        core_axis_name="c",
        subcore_axis_name="s",
        num_cores=_NUM_CORES,
        num_subcores=_NUM_SUBCORES,
    )

    @functools.partial(
        pl.kernel,
        mesh=mesh,
        compiler_params=pltpu.CompilerParams(use_tc_tiling_on_sc=False),
        out_type=jax.ShapeDtypeStruct((n_tokens, d), jnp.float32),
        scratch_types=[
            pltpu.VMEM((s, _CHUNK), jnp.int32),
            pltpu.VMEM((s, _CHUNK, d), jnp.float32),
            pltpu.VMEM((_CHUNK, d), jnp.float32),
            pltpu.SemaphoreType.DMA,
        ],
    )
    def sc_kernel(idx_hbm, tab_hbm, out_hbm, idx_v, rows_v, out_v, sem):
        wid = lax.axis_index("s") * _NUM_CORES + lax.axis_index("c")
        base0 = wid * per_w

        def chunk_body(g, carry):
            base = base0 + g * _CHUNK
            pltpu.sync_copy(idx_hbm.at[:, pl.ds(base, _CHUNK)], idx_v)
            copies = [
                pltpu.async_copy(tab_hbm.at[idx_v.at[j]], rows_v.at[j], sem)
                for j in range(s)
            ]
            for c in copies:
                c.wait()

            def tok_body(t, carry2):
                for k in range(vregs_per_tok):
                    col = pl.ds(k * _LANES, _LANES)
                    acc = rows_v[0, t, col] + rows_v[1, t, col]
                    for j in range(2, s):
                        acc = acc + rows_v[j, t, col]
                    out_v[t, col] = acc
                return carry2

            lax.fori_loop(0, _CHUNK, tok_body, 0, unroll=2)
            pltpu.sync_copy(out_v, out_hbm.at[pl.ds(base, _CHUNK), :])
            return carry

        lax.fori_loop(0, n_chunks, chunk_body, 0)

    return sc_kernel(flat_idx, tables)


def kernel(ctrl_tokens, embed_table, proj_w, proj_b):
    b, t, s = ctrl_tokens.shape
    v, d = embed_table.shape
    n = b * t

    tables = _precompute_tables(embed_table, proj_w, proj_b, row_tile=2000)

    idx = ctrl_tokens.reshape(n, s).astype(jnp.int32)
    offs = jnp.arange(s, dtype=jnp.int32) * v
    flat_idx = (idx + offs[None, :]).T

    out = _gather_sum(flat_idx, tables, n, s, d)
    return out.reshape(b, t, d)


# speedup vs baseline: 8.8407x; 2.0769x over previous
import functools

import jax
import jax.numpy as jnp
from jax import lax
from jax.experimental import pallas as pl
from jax.experimental.pallas import tpu as pltpu
from jax.experimental.pallas import tpu_sc as plsc

_NUM_CORES = 2
_NUM_SUBCORES = 16
_NW = _NUM_CORES * _NUM_SUBCORES
_LANES = 16

_CHUNK = 128


def _proj_body(x_ref, w_ref, b_ref, o_ref):
    o_ref[...] = (
        jnp.dot(x_ref[...], w_ref[...], preferred_element_type=jnp.float32)
        + b_ref[...]
    )


def _precompute_tables(embed_table, proj_w, proj_b, row_tile):
    v, d = embed_table.shape
    s = proj_w.shape[1] // d
    half = s // 2
    wt = proj_w.reshape(d, s, d).transpose(1, 2, 0)
    w2 = jnp.concatenate([wt[:half], wt[half:]], axis=2).reshape(half * d, 2 * d)
    bias = jnp.tile((proj_b / s).astype(jnp.float32), 2).reshape(1, 2 * d)
    n_row_tiles = v // row_tile
    return pl.pallas_call(
        _proj_body,
        grid=(n_row_tiles, half),
        in_specs=[
            pl.BlockSpec((row_tile, d), lambda k, a: (k, 0)),
            pl.BlockSpec((d, 2 * d), lambda k, a: (a, 0)),
            pl.BlockSpec((1, 2 * d), lambda k, a: (0, 0)),
        ],
        out_specs=pl.BlockSpec(
            (row_tile, 2 * d), lambda k, a: (a * n_row_tiles + k, 0)
        ),
        out_shape=jax.ShapeDtypeStruct((half * v, 2 * d), jnp.float32),
    )(embed_table, w2, bias)


def _gather_sum(flat_idx, tables, n_tokens, s, d):
    per_w = n_tokens // _NW
    n_chunks = per_w // _CHUNK
    assert n_chunks % 2 == 0

    mesh = plsc.VectorSubcoreMesh(
        core_axis_name="c",
        subcore_axis_name="s",
        num_cores=_NUM_CORES,
        num_subcores=_NUM_SUBCORES,
    )

    @functools.partial(
        pl.kernel,
        mesh=mesh,
        compiler_params=pltpu.CompilerParams(use_tc_tiling_on_sc=False),
        out_type=jax.ShapeDtypeStruct((n_tokens, d), jnp.float32),
        scratch_types=[
            pltpu.VMEM((s, per_w), jnp.int32),
            pltpu.VMEM((2, s, _CHUNK, d), jnp.float32),
            pltpu.VMEM((2, _CHUNK, d), jnp.float32),
            pltpu.SemaphoreType.DMA((2,)),
            pltpu.SemaphoreType.DMA((2,)),
        ],
    )
    def sc_kernel(idx_hbm, tab_hbm, out_hbm, idx_all, rows_v, out_v, gsem, osem):
        wid = lax.axis_index("s") * _NUM_CORES + lax.axis_index("c")
        base0 = wid * per_w
        pltpu.sync_copy(idx_hbm.at[:, pl.ds(base0, per_w)], idx_all)

        def gather_copy(buf, g, j):
            return pltpu.make_async_copy(
                tab_hbm.at[idx_all.at[j, pl.ds(g * _CHUNK, _CHUNK)]],
                rows_v.at[buf, j],
                gsem.at[buf],
            )

        def out_copy(buf, g):
            return pltpu.make_async_copy(
                out_v.at[buf],
                out_hbm.at[pl.ds(base0 + g * _CHUNK, _CHUNK), :],
                osem.at[buf],
            )

        def fire(buf, g):
            for j in range(s):
                gather_copy(buf, g, j).start()

        def compute(buf):
            def tok_body(t, carry):
                for k in range(d // _LANES):
                    col = pl.ds(k * _LANES, _LANES)
                    acc = rows_v[buf, 0, t, col] + rows_v[buf, 1, t, col]
                    for j in range(2, s):
                        acc = acc + rows_v[buf, j, t, col]
                    out_v[buf, t, col] = acc
                return carry

            lax.fori_loop(0, _CHUNK, tok_body, 0, unroll=4)

        fire(0, 0)

        def outer(i, carry):
            for buf in range(2):
                g = i * 2 + buf

                @pl.when(g + 1 < n_chunks)
                def _():
                    fire(1 - buf, g + 1)

                for j in range(s):
                    gather_copy(buf, g, j).wait()

                @pl.when(g >= 2)
                def _():
                    out_copy(buf, g - 2).wait()

                compute(buf)
                out_copy(buf, g).start()
            return carry

        lax.fori_loop(0, n_chunks // 2, outer, 0)
        out_copy(0, n_chunks - 2).wait()
        out_copy(1, n_chunks - 1).wait()

    return sc_kernel(flat_idx, tables)


def kernel(ctrl_tokens, embed_table, proj_w, proj_b):
    b, t, s = ctrl_tokens.shape
    v, d = embed_table.shape
    n = b * t

    tab2 = _precompute_tables(embed_table, proj_w, proj_b, row_tile=10000)
    tables = tab2.reshape(s * v, d)

    idx = ctrl_tokens.reshape(n, s).astype(jnp.int32)
    j = jnp.arange(s, dtype=jnp.int32)
    flat_idx = (2 * ((j % 2)[None, :] * v + idx) + (j // 2)[None, :]).T

    out = _gather_sum(flat_idx, tables, n, s, d)
    return out.reshape(b, t, d)


# speedup vs baseline: 9.1652x; 1.0367x over previous
import functools

import jax
import jax.numpy as jnp
from jax import lax
from jax.experimental import pallas as pl
from jax.experimental.pallas import tpu as pltpu
from jax.experimental.pallas import tpu_sc as plsc

_NUM_CORES = 2
_NUM_SUBCORES = 16
_NW = _NUM_CORES * _NUM_SUBCORES
_LANES = 16

_CHUNK = 128


def _proj_body(x_ref, w_ref, b_ref, o_ref):
    o_ref[...] = (
        jnp.dot(x_ref[...], w_ref[...], preferred_element_type=jnp.float32)
        + b_ref[...]
    )


def _precompute_tables(embed_table, proj_w, proj_b, row_tile):
    v, d = embed_table.shape
    s = proj_w.shape[1] // d
    half = s // 2
    wt = proj_w.reshape(d, s, d).transpose(1, 2, 0)
    w2 = jnp.concatenate([wt[:half], wt[half:]], axis=2).reshape(half * d, 2 * d)
    bias = jnp.tile((proj_b / s).astype(jnp.float32), 2).reshape(1, 2 * d)
    n_row_tiles = v // row_tile
    return pl.pallas_call(
        _proj_body,
        grid=(n_row_tiles, half),
        in_specs=[
            pl.BlockSpec((row_tile, d), lambda k, a: (k, 0)),
            pl.BlockSpec((d, 2 * d), lambda k, a: (a, 0)),
            pl.BlockSpec((1, 2 * d), lambda k, a: (0, 0)),
        ],
        out_specs=pl.BlockSpec(
            (row_tile, 2 * d), lambda k, a: (a * n_row_tiles + k, 0)
        ),
        out_shape=jax.ShapeDtypeStruct((half * v, 2 * d), jnp.float32),
    )(embed_table, w2, bias)


def _gather_sum(flat_idx, tables, n_tokens, s, d):
    per_w = n_tokens // _NW
    n_chunks = per_w // _CHUNK
    assert n_chunks % 2 == 0

    mesh = plsc.VectorSubcoreMesh(
        core_axis_name="c",
        subcore_axis_name="s",
        num_cores=_NUM_CORES,
        num_subcores=_NUM_SUBCORES,
    )

    @functools.partial(
        pl.kernel,
        mesh=mesh,
        compiler_params=pltpu.CompilerParams(use_tc_tiling_on_sc=False),
        out_type=jax.ShapeDtypeStruct((n_tokens // 2, 2 * d), jnp.float32),
        scratch_types=[
            pltpu.VMEM((s, per_w), jnp.int32),
            pltpu.VMEM((2, s, _CHUNK, d), jnp.float32),
            pltpu.VMEM((2, _CHUNK, d), jnp.float32),
            pltpu.SemaphoreType.DMA((2,)),
            pltpu.SemaphoreType.DMA((2,)),
        ],
    )
    def sc_kernel(idx_hbm, tab_hbm, out_hbm, idx_all, rows_v, out_v, gsem, osem):
        wid = lax.axis_index("s") * _NUM_CORES + lax.axis_index("c")
        base0 = wid * per_w
        half = base0 // (n_tokens // 2)
        row0 = base0 % (n_tokens // 2)
        pltpu.sync_copy(idx_hbm.at[:, pl.ds(base0, per_w)], idx_all)

        def gather_copy(buf, g, j):
            return pltpu.make_async_copy(
                tab_hbm.at[idx_all.at[j, pl.ds(g * _CHUNK, _CHUNK)]],
                rows_v.at[buf, j],
                gsem.at[buf],
            )

        def out_copy(buf, g):
            return pltpu.make_async_copy(
                out_v.at[buf],
                out_hbm.at[pl.ds(row0 + g * _CHUNK, _CHUNK), pl.ds(half * d, d)],
                osem.at[buf],
            )

        def fire(buf, g):
            for j in range(s):
                gather_copy(buf, g, j).start()

        def compute(buf):
            def tok_body(t, carry):
                for k in range(d // _LANES):
                    col = pl.ds(k * _LANES, _LANES)
                    acc = rows_v[buf, 0, t, col] + rows_v[buf, 1, t, col]
                    for j in range(2, s):
                        acc = acc + rows_v[buf, j, t, col]
                    out_v[buf, t, col] = acc
                return carry

            lax.fori_loop(0, _CHUNK, tok_body, 0, unroll=4)

        fire(0, 0)

        def outer(i, carry):
            for buf in range(2):
                g = i * 2 + buf

                @pl.when(g + 1 < n_chunks)
                def _():
                    fire(1 - buf, g + 1)

                for j in range(s):
                    gather_copy(buf, g, j).wait()

                @pl.when(g >= 2)
                def _():
                    out_copy(buf, g - 2).wait()

                compute(buf)
                out_copy(buf, g).start()
            return carry

        lax.fori_loop(0, n_chunks // 2, outer, 0)
        out_copy(0, n_chunks - 2).wait()
        out_copy(1, n_chunks - 1).wait()

    return sc_kernel(flat_idx, tables)


def _untile_body(x_ref, o_ref):
    h = pl.program_id(1)
    d = o_ref.shape[-1]

    @pl.when(h == 0)
    def _():
        o_ref[...] = x_ref[:, :d]

    @pl.when(h == 1)
    def _():
        o_ref[...] = x_ref[:, d:]


def _untile(packed, n, d, blk):
    m = (n // 2) // blk
    return pl.pallas_call(
        _untile_body,
        grid=(m, 2),
        in_specs=[pl.BlockSpec((blk, 2 * d), lambda i, h: (i, 0))],
        out_specs=pl.BlockSpec((blk, d), lambda i, h: (h * m + i, 0)),
        out_shape=jax.ShapeDtypeStruct((n, d), jnp.float32),
    )(packed)


def kernel(ctrl_tokens, embed_table, proj_w, proj_b):
    b, t, s = ctrl_tokens.shape
    v, d = embed_table.shape
    n = b * t

    tab2 = _precompute_tables(embed_table, proj_w, proj_b, row_tile=10000)
    tables = tab2.reshape(s * v, d)

    idx = ctrl_tokens.reshape(n, s).astype(jnp.int32)
    j = jnp.arange(s, dtype=jnp.int32)
    flat_idx = (2 * ((j % 2)[None, :] * v + idx) + (j // 2)[None, :]).T

    packed = _gather_sum(flat_idx, tables, n, s, d)
    out = _untile(packed, n, d, blk=6400)
    return out.reshape(b, t, d)
